# baseline (speedup 1.0000x reference)
import functools

import jax
import jax.numpy as jnp
from jax import lax
from jax.experimental import pallas as pl
from jax.experimental.pallas import tpu as pltpu
from jax.experimental.pallas import tpu_sc as plsc

EPS = 1e-5
MAXNORM = 1.0 - EPS

BATCH = 4096
NCAND = 50
DIM = 16
NW = 32
BPW = BATCH // NW
K = (BPW * NCAND) // 128


def _sc_gather_reduce(table, idx3d):
    mesh = plsc.VectorSubcoreMesh(core_axis_name="c", subcore_axis_name="s")

    @functools.partial(
        pl.kernel,
        mesh=mesh,
        out_type=jax.ShapeDtypeStruct((BATCH, 128), jnp.float32),
        scratch_types=[
            pltpu.VMEM((K, 128), jnp.int32),
            pltpu.VMEM((BPW * NCAND, DIM), jnp.float32),
            pltpu.VMEM((BPW, 128), jnp.float32),
            pltpu.SemaphoreType.DMA,
            pltpu.SemaphoreType.DMA,
        ],
        compiler_params=pltpu.CompilerParams(
            use_tc_tiling_on_sc=False, needs_layout_passes=False
        ),
    )
    def k(table_hbm, idx_hbm, out_hbm, idx_v, rows_v, out_v, sem, sem2):
        wid = lax.axis_index("s") * 2 + lax.axis_index("c")
        pltpu.sync_copy(idx_hbm.at[wid], idx_v)

        def remap(j, carry):
            for kk in range(8):
                v = idx_v[j, pl.ds(kk * 16, 16)]
                q = v & (CB - 1)
                idx_v[j, pl.ds(kk * 16, 16)] = (
                    (v - q) + ((q & (SLAB - 1)) << 3) + (q >> 14)
                )
            return carry

        lax.fori_loop(0, K, remap, 0)

        def fire(j, carry):
            pltpu.async_copy(
                table_hbm.at[idx_v.at[j]], rows_v.at[pl.ds(j * 128, 128)], sem
            )
            return carry

        def fire2(j, carry):
            pltpu.async_copy(
                table_hbm.at[idx_v.at[j]], rows_v.at[pl.ds(j * 128, 128)], sem2
            )
            return carry

        lax.fori_loop(0, K // 2, fire, 0)
        lax.fori_loop(K // 2, K, fire2, 0)

        def drain(j, carry):
            pltpu.make_async_copy(
                table_hbm.at[pl.ds(0, 128)], rows_v.at[pl.ds(0, 128)], sem
            ).wait()
            return carry

        def drain2(j, carry):
            pltpu.make_async_copy(
                table_hbm.at[pl.ds(0, 128)], rows_v.at[pl.ds(0, 128)], sem2
            ).wait()
            return carry

        iota = lax.iota(jnp.int32, 16)
        dvec = [(iota + s) & (DIM - 1) for s in range(DIM)]
        ngrp = 4

        def body(b, carry):
            row0 = b * NCAND
            row0v = iota * 0 + row0
            u_s = [plsc.load_gather(rows_v, [row0v, dvec[s]])
                   for s in range(DIM)]
            rowv = [
                row0 + jnp.minimum(g * 16 + iota, NCAND - 1) for g in range(ngrp)
            ]
            for g in range(ngrp):
                acc_vv = jnp.zeros((16,), jnp.float32)
                acc_uv = jnp.zeros((16,), jnp.float32)
                for s in range(DIM):
                    vals = plsc.load_gather(rows_v, [rowv[g], dvec[s]])
                    acc_vv = acc_vv + vals * vals
                    acc_uv = acc_uv + vals * u_s[s]
                out_v[b, pl.ds(g * 16, 16)] = acc_vv
                out_v[b, pl.ds(64 + g * 16, 16)] = acc_uv
            return carry

        lax.fori_loop(0, K // 2, drain, 0)
        lax.fori_loop(0, BPW // 2, body, 0)
        lax.fori_loop(0, K - K // 2, drain2, 0)
        lax.fori_loop(BPW // 2, BPW, body, 0)
        pltpu.sync_copy(out_v, out_hbm.at[pl.ds(wid * BPW, BPW)])

    return k(table, idx3d)


CB = 131072
SLAB = CB // 8
NBLK = 8
VPAD = NBLK * CB


def _repack_body(x_ref, out_ref):
    x = x_ref[...]
    xs = jnp.concatenate(
        [lax.slice(x, (0, SLAB * m), (DIM, SLAB * (m + 1))) for m in range(8)],
        axis=0,
    )
    eye = (
        lax.broadcasted_iota(jnp.int32, (128, 128), 0)
        == lax.broadcasted_iota(jnp.int32, (128, 128), 1)
    ).astype(jnp.float32)
    out_ref[...] = lax.dot_general(
        xs, eye, (((0,), (0,)), ((), ())), preferred_element_type=jnp.float32
    )


def _tc_repack(table_t):
    return pl.pallas_call(
        _repack_body,
        grid=(NBLK,),
        in_specs=[pl.BlockSpec((DIM, CB), lambda i: (0, i))],
        out_specs=pl.BlockSpec((CB // 8, 128), lambda i: (i, 0)),
        out_shape=jax.ShapeDtypeStruct((VPAD // 8, 128), jnp.float32),
    )(table_t)


def _tc_body(x_ref, out_ref):
    x = x_ref[...]
    vv = x[:, 0:64]
    uv = x[:, 64:128]
    uu = vv[:, 0:1]
    norm_u = jnp.sqrt(uu)
    su = jnp.where(norm_u > MAXNORM, MAXNORM / jnp.maximum(norm_u, EPS), 1.0)
    norm_v = jnp.sqrt(vv)
    sv = jnp.where(norm_v > MAXNORM, MAXNORM / jnp.maximum(norm_v, EPS), 1.0)
    uu_n = uu * su * su
    vv_n = vv * sv * sv
    uv_n = uv * su * sv
    duv = uu_n + vv_n - 2.0 * uv_n
    alpha = jnp.clip(1.0 - uu_n, EPS, None)
    beta = jnp.clip(1.0 - vv_n, EPS, None)
    gamma = 1.0 + 2.0 * duv / (alpha * beta)
    g = jnp.clip(gamma, 1.0 + EPS, None)
    fval = jnp.log(g + jnp.sqrt((g - 1.0) * (g + 1.0)))
    out_ref[...] = fval[:, 1:NCAND]


def _tc_distance(x):
    return pl.pallas_call(
        _tc_body,
        out_shape=jax.ShapeDtypeStruct((BATCH, NCAND - 1), jnp.float32),
    )(x)


def kernel(inputs, table):
    idx3d = inputs.reshape(NW, K, 128)
    tab_lin = _tc_repack(table.T).reshape(VPAD, DIM)
    packed = _sc_gather_reduce(tab_lin, idx3d)
    return _tc_distance(packed)


# speedup vs baseline: 1.0193x; 1.0193x over previous
import functools

import jax
import jax.numpy as jnp
from jax import lax
from jax.experimental import pallas as pl
from jax.experimental.pallas import tpu as pltpu
from jax.experimental.pallas import tpu_sc as plsc

EPS = 1e-5
MAXNORM = 1.0 - EPS

BATCH = 4096
NCAND = 50
DIM = 16
NW = 32
BPW = BATCH // NW
K = (BPW * NCAND) // 128


def _sc_gather_reduce(table, idx3d):
    mesh = plsc.VectorSubcoreMesh(core_axis_name="c", subcore_axis_name="s")

    @functools.partial(
        pl.kernel,
        mesh=mesh,
        out_type=jax.ShapeDtypeStruct((BATCH, 128), jnp.float32),
        scratch_types=[
            pltpu.VMEM((K, 128), jnp.int32),
            pltpu.VMEM((BPW * NCAND, DIM), jnp.float32),
            pltpu.VMEM((BPW, 128), jnp.float32),
            pltpu.SemaphoreType.DMA,
            pltpu.SemaphoreType.DMA,
        ],
        compiler_params=pltpu.CompilerParams(
            use_tc_tiling_on_sc=False, needs_layout_passes=False
        ),
    )
    def k(table_hbm, idx_hbm, out_hbm, idx_v, rows_v, out_v, sem, sem2):
        wid = lax.axis_index("s") * 2 + lax.axis_index("c")
        pltpu.sync_copy(idx_hbm.at[wid], idx_v)

        def remap(j, carry):
            for kk in range(8):
                v = idx_v[j, pl.ds(kk * 16, 16)]
                q = v & (CB - 1)
                idx_v[j, pl.ds(kk * 16, 16)] = (
                    (v - q) + ((q & (SLAB - 1)) << 3) + (q >> 14)
                )
            return carry

        lax.fori_loop(0, K, remap, 0)

        def fire(j, carry):
            pltpu.async_copy(
                table_hbm.at[idx_v.at[j]], rows_v.at[pl.ds(j * 128, 128)], sem
            )
            return carry

        def fire2(j, carry):
            pltpu.async_copy(
                table_hbm.at[idx_v.at[j]], rows_v.at[pl.ds(j * 128, 128)], sem2
            )
            return carry

        lax.fori_loop(0, K // 2, fire, 0)
        lax.fori_loop(K // 2, K, fire2, 0)

        def drain(j, carry):
            pltpu.make_async_copy(
                table_hbm.at[pl.ds(0, 128)], rows_v.at[pl.ds(0, 128)], sem
            ).wait()
            return carry

        def drain2(j, carry):
            pltpu.make_async_copy(
                table_hbm.at[pl.ds(0, 128)], rows_v.at[pl.ds(0, 128)], sem2
            ).wait()
            return carry

        iota = lax.iota(jnp.int32, 16)
        dvec = [(iota + s) & (DIM - 1) for s in range(DIM)]
        ngrp = 4

        def body(b, carry):
            row0 = b * NCAND
            row0v = iota * 0 + row0
            u_s = [plsc.load_gather(rows_v, [row0v, dvec[s]])
                   for s in range(DIM)]
            rowv = [
                row0 + jnp.minimum(g * 16 + iota, NCAND - 1) for g in range(ngrp)
            ]
            for g in range(ngrp):
                a_vv = [jnp.zeros((16,), jnp.float32) for _ in range(2)]
                a_uv = [jnp.zeros((16,), jnp.float32) for _ in range(2)]
                for s in range(DIM):
                    vals = plsc.load_gather(rows_v, [rowv[g], dvec[s]])
                    a_vv[s % 2] = a_vv[s % 2] + vals * vals
                    a_uv[s % 2] = a_uv[s % 2] + vals * u_s[s]
                out_v[b, pl.ds(g * 16, 16)] = a_vv[0] + a_vv[1]
                out_v[b, pl.ds(64 + g * 16, 16)] = a_uv[0] + a_uv[1]
            return carry

        lax.fori_loop(0, K // 2, drain, 0)
        lax.fori_loop(0, BPW // 2, body, 0)
        lax.fori_loop(0, K - K // 2, drain2, 0)
        lax.fori_loop(BPW // 2, BPW, body, 0)
        pltpu.sync_copy(out_v, out_hbm.at[pl.ds(wid * BPW, BPW)])

    return k(table, idx3d)


CB = 131072
SLAB = CB // 8
NBLK = 8
VPAD = NBLK * CB


def _repack_body(x_ref, out_ref):
    x = x_ref[...]
    xs = jnp.concatenate(
        [lax.slice(x, (0, SLAB * m), (DIM, SLAB * (m + 1))) for m in range(8)],
        axis=0,
    )
    eye = (
        lax.broadcasted_iota(jnp.int32, (128, 128), 0)
        == lax.broadcasted_iota(jnp.int32, (128, 128), 1)
    ).astype(jnp.float32)
    out_ref[...] = lax.dot_general(
        xs, eye, (((0,), (0,)), ((), ())), preferred_element_type=jnp.float32
    )


def _tc_repack(table_t):
    return pl.pallas_call(
        _repack_body,
        grid=(NBLK,),
        in_specs=[pl.BlockSpec((DIM, CB), lambda i: (0, i))],
        out_specs=pl.BlockSpec((CB // 8, 128), lambda i: (i, 0)),
        out_shape=jax.ShapeDtypeStruct((VPAD // 8, 128), jnp.float32),
    )(table_t)


def _tc_body(x_ref, out_ref):
    x = x_ref[...]
    vv = x[:, 0:64]
    uv = x[:, 64:128]
    uu = vv[:, 0:1]
    norm_u = jnp.sqrt(uu)
    su = jnp.where(norm_u > MAXNORM, MAXNORM / jnp.maximum(norm_u, EPS), 1.0)
    norm_v = jnp.sqrt(vv)
    sv = jnp.where(norm_v > MAXNORM, MAXNORM / jnp.maximum(norm_v, EPS), 1.0)
    uu_n = uu * su * su
    vv_n = vv * sv * sv
    uv_n = uv * su * sv
    duv = uu_n + vv_n - 2.0 * uv_n
    alpha = jnp.clip(1.0 - uu_n, EPS, None)
    beta = jnp.clip(1.0 - vv_n, EPS, None)
    gamma = 1.0 + 2.0 * duv / (alpha * beta)
    g = jnp.clip(gamma, 1.0 + EPS, None)
    fval = jnp.log(g + jnp.sqrt((g - 1.0) * (g + 1.0)))
    out_ref[...] = fval[:, 1:NCAND]


def _tc_distance(x):
    return pl.pallas_call(
        _tc_body,
        out_shape=jax.ShapeDtypeStruct((BATCH, NCAND - 1), jnp.float32),
    )(x)


def kernel(inputs, table):
    idx3d = inputs.reshape(NW, K, 128)
    tab_lin = _tc_repack(table.T).reshape(VPAD, DIM)
    packed = _sc_gather_reduce(tab_lin, idx3d)
    return _tc_distance(packed)


# speedup vs baseline: 1.0586x; 1.0386x over previous
import functools

import jax
import jax.numpy as jnp
from jax import lax
from jax.experimental import pallas as pl
from jax.experimental.pallas import tpu as pltpu
from jax.experimental.pallas import tpu_sc as plsc

EPS = 1e-5
MAXNORM = 1.0 - EPS

BATCH = 4096
NCAND = 50
DIM = 16
NW = 32
BPW = BATCH // NW
K = (BPW * NCAND) // 128


def _sc_gather_reduce(table, idx3d):
    mesh = plsc.VectorSubcoreMesh(core_axis_name="c", subcore_axis_name="s")

    @functools.partial(
        pl.kernel,
        mesh=mesh,
        out_type=jax.ShapeDtypeStruct((BATCH, 128), jnp.float32),
        scratch_types=[
            pltpu.VMEM((K, 128), jnp.int32),
            pltpu.VMEM((BPW * NCAND, DIM), jnp.float32),
            pltpu.VMEM((BPW, 128), jnp.float32),
            pltpu.SemaphoreType.DMA,
            pltpu.SemaphoreType.DMA,
        ],
        compiler_params=pltpu.CompilerParams(
            use_tc_tiling_on_sc=False, needs_layout_passes=False
        ),
    )
    def k(table_hbm, idx_hbm, out_hbm, idx_v, rows_v, out_v, sem, sem2):
        wid = lax.axis_index("s") * 2 + lax.axis_index("c")
        pltpu.sync_copy(idx_hbm.at[wid], idx_v)

        def remap(j, carry):
            for kk in range(8):
                v = idx_v[j, pl.ds(kk * 16, 16)]
                q = v & (CB - 1)
                idx_v[j, pl.ds(kk * 16, 16)] = (
                    (v - q) + ((q & (SLAB - 1)) << 3) + (q >> 14)
                )
            return carry

        lax.fori_loop(0, K, remap, 0)

        def fire(j, carry):
            pltpu.async_copy(
                table_hbm.at[idx_v.at[j]], rows_v.at[pl.ds(j * 128, 128)], sem
            )
            return carry

        def fire2(j, carry):
            pltpu.async_copy(
                table_hbm.at[idx_v.at[j]], rows_v.at[pl.ds(j * 128, 128)], sem2
            )
            return carry

        lax.fori_loop(0, K // 2, fire, 0)
        lax.fori_loop(K // 2, K, fire2, 0)

        def drain(j, carry):
            pltpu.make_async_copy(
                table_hbm.at[pl.ds(0, 128)], rows_v.at[pl.ds(0, 128)], sem
            ).wait()
            return carry

        def drain2(j, carry):
            pltpu.make_async_copy(
                table_hbm.at[pl.ds(0, 128)], rows_v.at[pl.ds(0, 128)], sem2
            ).wait()
            return carry

        iota = lax.iota(jnp.int32, 16)
        dvec = [(iota + s) & (DIM - 1) for s in range(DIM)]
        ngrp = 4

        def body(b, carry):
            row0 = b * NCAND
            row0v = iota * 0 + row0
            u_s = [plsc.load_gather(rows_v, [row0v, dvec[s]])
                   for s in range(DIM)]
            rowv = [
                row0 + jnp.minimum(g * 16 + iota, NCAND - 1) for g in range(ngrp)
            ]
            for g in range(ngrp):
                a_vv = [jnp.zeros((16,), jnp.float32) for _ in range(2)]
                a_uv = [jnp.zeros((16,), jnp.float32) for _ in range(2)]
                for s in range(DIM):
                    vals = plsc.load_gather(rows_v, [rowv[g], dvec[s]])
                    a_vv[s % 2] = a_vv[s % 2] + vals * vals
                    a_uv[s % 2] = a_uv[s % 2] + vals * u_s[s]
                out_v[b, pl.ds(g * 16, 16)] = a_vv[0] + a_vv[1]
                out_v[b, pl.ds(64 + g * 16, 16)] = a_uv[0] + a_uv[1]
            return carry

        lax.fori_loop(0, K // 2, drain, 0)
        lax.fori_loop(0, BPW // 2, body, 0)
        lax.fori_loop(0, K - K // 2, drain2, 0)
        lax.fori_loop(BPW // 2, BPW, body, 0)
        pltpu.sync_copy(out_v, out_hbm.at[pl.ds(wid * BPW, BPW)])

    return k(table, idx3d)


CB = 131072
SLAB = CB // 8
NBLK = 8
VPAD = NBLK * CB


def _repack_body(x_ref, out_ref):
    x = x_ref[...]
    xs = jnp.concatenate(
        [lax.slice(x, (0, SLAB * m), (DIM, SLAB * (m + 1))) for m in range(8)],
        axis=0,
    )
    eye = (
        lax.broadcasted_iota(jnp.int32, (128, 128), 0)
        == lax.broadcasted_iota(jnp.int32, (128, 128), 1)
    ).astype(jnp.float32)
    out_ref[...] = lax.dot_general(
        xs, eye, (((0,), (0,)), ((), ())), preferred_element_type=jnp.float32
    )


def _tc_repack(table_t):
    return pl.pallas_call(
        _repack_body,
        grid=(NBLK,),
        in_specs=[pl.BlockSpec((DIM, CB), lambda i: (0, i))],
        out_specs=pl.BlockSpec((CB // 8, 128), lambda i: (i, 0)),
        out_shape=jax.ShapeDtypeStruct((VPAD // 8, 128), jnp.float32),
    )(table_t)


def _tc_body(x_ref, out_ref):
    x = x_ref[...]
    vv = x[:, 0:64]
    uv = x[:, 64:128]
    uu = vv[:, 0:1]
    norm_u = jnp.sqrt(uu)
    su = jnp.where(norm_u > MAXNORM, MAXNORM / jnp.maximum(norm_u, EPS), 1.0)
    norm_v = jnp.sqrt(vv)
    sv = jnp.where(norm_v > MAXNORM, MAXNORM / jnp.maximum(norm_v, EPS), 1.0)
    uu_n = uu * su * su
    vv_n = vv * sv * sv
    uv_n = uv * su * sv
    duv = uu_n + vv_n - 2.0 * uv_n
    alpha = jnp.clip(1.0 - uu_n, EPS, None)
    beta = jnp.clip(1.0 - vv_n, EPS, None)
    gamma = 1.0 + 2.0 * duv / (alpha * beta)
    g = jnp.clip(gamma, 1.0 + EPS, None)
    fval = jnp.log(g + jnp.sqrt((g - 1.0) * (g + 1.0)))
    ft = jnp.swapaxes(fval, 0, 1)
    out_ref[...] = ft[1:NCAND, :]


def _tc_distance(x):
    return pl.pallas_call(
        _tc_body,
        out_shape=jax.ShapeDtypeStruct((NCAND - 1, BATCH), jnp.float32),
    )(x)


def kernel(inputs, table):
    idx3d = inputs.reshape(NW, K, 128)
    tab_lin = _tc_repack(table.T).reshape(VPAD, DIM)
    packed = _sc_gather_reduce(tab_lin, idx3d)
    return _tc_distance(packed).T
